# Initial kernel scaffold; baseline (speedup 1.0000x reference)
import jax
import jax.numpy as jnp
from jax.experimental import pallas as pl


def kernel(x, Wq, bq, Wk, bk, Wv, bv, Wo, bo, W1, b1, W2, b2, gamma, beta):
    raise NotImplementedError("write your pallas kernel here")


# speedup vs baseline: 2.7149x; 2.7149x over previous
import math

import jax
import jax.numpy as jnp
import numpy as np
from jax.experimental import pallas as pl
from jax.experimental.pallas import tpu as pltpu

_B, _L, _DM, _H, _DH = 2, 2048, 1024, 16, 64
_BL = _B * _L
_U = 40
_KT = 4
_KC = _L // _KT
_BH = _B * _H

_IDX = np.asarray(jax.random.randint(jax.random.key(42), (_L, _U), 0, _L))
_cnt = np.zeros((_L, _L), np.float32)
np.add.at(_cnt, (np.arange(_L)[:, None], _IDX), 1.0)
_CNT_T = np.ascontiguousarray(_cnt.T)
_NEG_T = np.where(_CNT_T > 0, 0.0, -1e30).astype(np.float32)
del _cnt


def _ln(t):
    mu = jnp.mean(t, axis=1, keepdims=True)
    xc = t - mu
    var = jnp.mean(xc * xc, axis=1, keepdims=True)
    return xc * jax.lax.rsqrt(var + 1e-5)


def _mm_bias_kernel(x_ref, w_ref, b_ref, o_ref):
    o_ref[...] = jnp.dot(x_ref[...].astype(jnp.bfloat16), w_ref[...],
                         preferred_element_type=jnp.float32) + b_ref[...]


def _mstats_kernel(q_ref, k_ref, cnt_ref, neg_ref, m_ref, accmax, accsum):
    kt = pl.program_id(0)
    b = pl.program_id(1)
    h = pl.program_id(2)
    st = jax.lax.dot_general(
        k_ref[0].astype(jnp.bfloat16), q_ref[0].astype(jnp.bfloat16),
        (((1,), (1,)), ((), ())), preferred_element_type=jnp.float32)
    tmax = jnp.max(st + neg_ref[...], axis=0, keepdims=True)
    tsum = jnp.sum(st * cnt_ref[...], axis=0, keepdims=True)
    row = b * _H + h
    first = kt == 0
    pm = accmax[pl.ds(row, 1), :]
    ps = accsum[pl.ds(row, 1), :]
    nm = jnp.where(first, tmax, jnp.maximum(pm, tmax))
    ns = jnp.where(first, tsum, ps + tsum)
    accmax[pl.ds(row, 1), :] = nm
    accsum[pl.ds(row, 1), :] = ns

    @pl.when(kt == _KT - 1)
    def _():
        m_ref[pl.ds(row, 1), :] = nm - ns * (1.0 / _L)


def _topk_kernel(m_ref, ti_ref):
    X = m_ref[...]
    io = jax.lax.broadcasted_iota(jnp.int32, (_BH, _L), 1)
    for u in range(_U):
        mx = jnp.max(X, axis=1, keepdims=True)
        cand = jnp.where(X >= mx, io, _L)
        am = jnp.min(cand, axis=1, keepdims=True)
        ti_ref[:, u:u + 1] = am
        X = jnp.where(io == am, -jnp.inf, X)


def _attn_kernel(q_ref, k_ref, v_ref, tr_ref, tc_ref, o_ref):
    qb = q_ref[0].astype(jnp.bfloat16)
    kb = k_ref[0].astype(jnp.bfloat16)
    vb = v_ref[0].astype(jnp.bfloat16)
    ti_row = tr_ref[0]
    ti_col = tc_ref[0]
    io_l = jax.lax.broadcasted_iota(jnp.int32, (_L, _U), 0)
    io_u = jax.lax.broadcasted_iota(jnp.int32, (_U, _L), 1)
    E = (io_l == ti_row).astype(jnp.bfloat16)
    Et = (io_u == ti_col).astype(jnp.bfloat16)
    qr = jnp.dot(Et, qb, preferred_element_type=jnp.float32)
    sc = jax.lax.dot_general(
        qr.astype(jnp.bfloat16), kb, (((1,), (1,)), ((), ())),
        preferred_element_type=jnp.float32) * (1.0 / math.sqrt(_DH))
    mx = jnp.max(sc, axis=1, keepdims=True)
    ex = jnp.exp(sc - mx)
    attn = (ex / jnp.sum(ex, axis=1, keepdims=True)).astype(jnp.bfloat16)
    av = jnp.dot(attn, vb, preferred_element_type=jnp.float32)
    vmean = jnp.mean(v_ref[0], axis=0, keepdims=True)
    delta = (av - vmean).astype(jnp.bfloat16)
    o_ref[0] = jnp.dot(E, delta, preferred_element_type=jnp.float32) + vmean


def _wo_addnorm_kernel(c_ref, w_ref, b_ref, h_ref, o_ref):
    t = jnp.dot(c_ref[...].astype(jnp.bfloat16), w_ref[...],
                preferred_element_type=jnp.float32)
    o_ref[...] = _ln(t + b_ref[...] + h_ref[...])


def _ffn1_kernel(y_ref, w_ref, b_ref, o_ref):
    t = jnp.dot(y_ref[...].astype(jnp.bfloat16), w_ref[...],
                preferred_element_type=jnp.float32)
    o_ref[...] = jnp.maximum(t + b_ref[...], 0.0).astype(jnp.bfloat16)


def _ffn2_kernel(r_ref, w_ref, b_ref, y_ref, o_ref):
    t = jnp.dot(r_ref[...], w_ref[...], preferred_element_type=jnp.float32)
    o_ref[...] = _ln(t + b_ref[...] + y_ref[...])


def _ffn2_final_kernel(r_ref, w_ref, b_ref, y_ref, g_ref, bb_ref, o_ref):
    t = jnp.dot(r_ref[...], w_ref[...], preferred_element_type=jnp.float32)
    hn = _ln(t + b_ref[...] + y_ref[...])
    o_ref[...] = _ln(hn) * g_ref[...] + bb_ref[...]


def kernel(x, Wq, bq, Wk, bk, Wv, bv, Wo, bo, W1, b1, W2, b2, gamma, beta):
    f32 = jnp.float32
    bf16 = jnp.bfloat16
    h = x.reshape(_BL, _DM)
    cnt_t = jnp.asarray(_CNT_T)
    neg_t = jnp.asarray(_NEG_T)

    out = None
    for i in range(2):
        wqkv = jnp.concatenate([Wq[i], Wk[i], Wv[i]], axis=1).astype(bf16)
        bqkv = jnp.concatenate([bq[i], bk[i], bv[i]])[None, :]

        qkv = pl.pallas_call(
            _mm_bias_kernel,
            grid=(8,),
            in_specs=[
                pl.BlockSpec((512, _DM), lambda m: (m, 0)),
                pl.BlockSpec((_DM, 3 * _DM), lambda m: (0, 0)),
                pl.BlockSpec((1, 3 * _DM), lambda m: (0, 0)),
            ],
            out_specs=pl.BlockSpec((512, 3 * _DM), lambda m: (m, 0)),
            out_shape=jax.ShapeDtypeStruct((_BL, 3 * _DM), f32),
        )(h, wqkv, bqkv)

        def _heads(a):
            return (a.reshape(_B, _L, _H, _DH).transpose(0, 2, 1, 3)
                    .reshape(_BH, _L, _DH))

        Qh = _heads(qkv[:, :_DM])
        Kh = _heads(qkv[:, _DM:2 * _DM])
        Vh = _heads(qkv[:, 2 * _DM:])

        M = pl.pallas_call(
            _mstats_kernel,
            grid=(_KT, _B, _H),
            in_specs=[
                pl.BlockSpec((1, _L, _DH), lambda kt, b, hh: (b * _H + hh, 0, 0)),
                pl.BlockSpec((1, _KC, _DH), lambda kt, b, hh: (b * _H + hh, kt, 0)),
                pl.BlockSpec((_KC, _L), lambda kt, b, hh: (kt, 0)),
                pl.BlockSpec((_KC, _L), lambda kt, b, hh: (kt, 0)),
            ],
            out_specs=pl.BlockSpec((_BH, _L), lambda kt, b, hh: (0, 0)),
            out_shape=jax.ShapeDtypeStruct((_BH, _L), f32),
            scratch_shapes=[pltpu.VMEM((_BH, _L), f32),
                            pltpu.VMEM((_BH, _L), f32)],
        )(Qh, Kh, cnt_t, neg_t)

        TI = pl.pallas_call(
            _topk_kernel,
            in_specs=[pl.BlockSpec((_BH, _L), lambda: (0, 0))],
            out_specs=pl.BlockSpec((_BH, _U), lambda: (0, 0)),
            out_shape=jax.ShapeDtypeStruct((_BH, _U), jnp.int32),
        )(M)

        ti_row = TI.reshape(_BH, 1, _U)
        ti_col = TI.reshape(_BH, _U, 1)

        ctxh = pl.pallas_call(
            _attn_kernel,
            grid=(_BH,),
            in_specs=[
                pl.BlockSpec((1, _L, _DH), lambda g: (g, 0, 0)),
                pl.BlockSpec((1, _L, _DH), lambda g: (g, 0, 0)),
                pl.BlockSpec((1, _L, _DH), lambda g: (g, 0, 0)),
                pl.BlockSpec((1, 1, _U), lambda g: (g, 0, 0)),
                pl.BlockSpec((1, _U, 1), lambda g: (g, 0, 0)),
            ],
            out_specs=pl.BlockSpec((1, _L, _DH), lambda g: (g, 0, 0)),
            out_shape=jax.ShapeDtypeStruct((_BH, _L, _DH), f32),
        )(Qh, Kh, Vh, ti_row, ti_col)
        ctx = (ctxh.reshape(_B, _H, _L, _DH).transpose(0, 2, 1, 3)
               .reshape(_BL, _DM))

        Y = pl.pallas_call(
            _wo_addnorm_kernel,
            grid=(16,),
            in_specs=[
                pl.BlockSpec((256, _DM), lambda m: (m, 0)),
                pl.BlockSpec((_DM, _DM), lambda m: (0, 0)),
                pl.BlockSpec((1, _DM), lambda m: (0, 0)),
                pl.BlockSpec((256, _DM), lambda m: (m, 0)),
            ],
            out_specs=pl.BlockSpec((256, _DM), lambda m: (m, 0)),
            out_shape=jax.ShapeDtypeStruct((_BL, _DM), f32),
        )(ctx, Wo[i].astype(bf16), bo[i][None, :], h)

        R = pl.pallas_call(
            _ffn1_kernel,
            grid=(8,),
            in_specs=[
                pl.BlockSpec((512, _DM), lambda m: (m, 0)),
                pl.BlockSpec((_DM, 4096), lambda m: (0, 0)),
                pl.BlockSpec((1, 4096), lambda m: (0, 0)),
            ],
            out_specs=pl.BlockSpec((512, 4096), lambda m: (m, 0)),
            out_shape=jax.ShapeDtypeStruct((_BL, 4096), bf16),
        )(Y, W1[i].astype(bf16), b1[i][None, :])

        if i == 0:
            h = pl.pallas_call(
                _ffn2_kernel,
                grid=(16,),
                in_specs=[
                    pl.BlockSpec((256, 4096), lambda m: (m, 0)),
                    pl.BlockSpec((4096, _DM), lambda m: (0, 0)),
                    pl.BlockSpec((1, _DM), lambda m: (0, 0)),
                    pl.BlockSpec((256, _DM), lambda m: (m, 0)),
                ],
                out_specs=pl.BlockSpec((256, _DM), lambda m: (m, 0)),
                out_shape=jax.ShapeDtypeStruct((_BL, _DM), f32),
            )(R, W2[i].astype(bf16), b2[i][None, :], Y)
        else:
            out = pl.pallas_call(
                _ffn2_final_kernel,
                grid=(16,),
                in_specs=[
                    pl.BlockSpec((256, 4096), lambda m: (m, 0)),
                    pl.BlockSpec((4096, _DM), lambda m: (0, 0)),
                    pl.BlockSpec((1, _DM), lambda m: (0, 0)),
                    pl.BlockSpec((256, _DM), lambda m: (m, 0)),
                    pl.BlockSpec((1, _DM), lambda m: (0, 0)),
                    pl.BlockSpec((1, _DM), lambda m: (0, 0)),
                ],
                out_specs=pl.BlockSpec((256, _DM), lambda m: (m, 0)),
                out_shape=jax.ShapeDtypeStruct((_BL, _DM), f32),
            )(R, W2[i].astype(bf16), b2[i][None, :], Y,
              gamma[None, :], beta[None, :])
    return out.reshape(_B, _L, _DM)


# speedup vs baseline: 3.7928x; 1.3970x over previous
import math

import jax
import jax.numpy as jnp
import numpy as np
from jax.experimental import pallas as pl
from jax.experimental.pallas import tpu as pltpu

_B, _L, _DM, _H, _DH = 2, 2048, 1024, 16, 64
_BL = _B * _L
_U = 40
_KT = 4
_KC = _L // _KT
_BH = _B * _H
_DP = 128
_HP = _H * _DP

_IDX = np.asarray(jax.random.randint(jax.random.key(42), (_L, _U), 0, _L))
_cnt = np.zeros((_L, _L), np.float32)
np.add.at(_cnt, (np.arange(_L)[:, None], _IDX), 1.0)
_CNT_T = np.asarray(jnp.asarray(np.ascontiguousarray(_cnt.T), jnp.bfloat16))
_NEG_T = np.asarray(jnp.asarray(
    np.where(_cnt.T > 0, 0.0, -1e30).astype(np.float32), jnp.bfloat16))
del _cnt


def _ln(t):
    mu = jnp.mean(t, axis=1, keepdims=True)
    xc = t - mu
    var = jnp.mean(xc * xc, axis=1, keepdims=True)
    return xc * jax.lax.rsqrt(var + 1e-5)


def _mm_bias_kernel(x_ref, w_ref, b_ref, o_ref):
    t = jnp.dot(x_ref[...].astype(jnp.bfloat16), w_ref[...],
                preferred_element_type=jnp.float32) + b_ref[...]
    o_ref[...] = t.astype(jnp.bfloat16)


def _mstats_kernel(q_ref, k_ref, cnt_ref, neg_ref, m_ref, accmax, accsum):
    kt = pl.program_id(0)
    b = pl.program_id(1)
    h = pl.program_id(2)
    st = jax.lax.dot_general(
        k_ref[...], q_ref[...],
        (((1,), (1,)), ((), ())), preferred_element_type=jnp.float32
        ).astype(jnp.bfloat16)
    tmax = jnp.max(st + neg_ref[...], axis=0, keepdims=True
                   ).astype(jnp.float32)
    tsum = jnp.sum(st * cnt_ref[...], axis=0, keepdims=True
                   ).astype(jnp.float32)
    row = b * _H + h
    first = kt == 0
    pm = accmax[pl.ds(row, 1), :]
    ps = accsum[pl.ds(row, 1), :]
    nm = jnp.where(first, tmax, jnp.maximum(pm, tmax))
    ns = jnp.where(first, tsum, ps + tsum)
    accmax[pl.ds(row, 1), :] = nm
    accsum[pl.ds(row, 1), :] = ns

    @pl.when(kt == _KT - 1)
    def _():
        m_ref[pl.ds(row, 1), :] = nm - ns * (1.0 / _L)


def _topk_kernel(m_ref, ti_ref):
    X = m_ref[...]
    io = jax.lax.broadcasted_iota(jnp.int32, (_BH, _L), 1)
    for u in range(_U):
        mx = jnp.max(X, axis=1, keepdims=True)
        cand = jnp.where(X >= mx, io, _L)
        am = jnp.min(cand, axis=1, keepdims=True)
        ti_ref[:, u:u + 1] = am
        X = jnp.where(io == am, -jnp.inf, X)


def _attn_kernel(q_ref, k_ref, v_ref, tr_ref, tc_ref, o_ref):
    qb = q_ref[...]
    kb = k_ref[...]
    vb = v_ref[...]
    ti_row = tr_ref[0]
    ti_col = tc_ref[0]
    io_l = jax.lax.broadcasted_iota(jnp.int32, (_L, _U), 0)
    io_u = jax.lax.broadcasted_iota(jnp.int32, (_U, _L), 1)
    E = (io_l == ti_row).astype(jnp.bfloat16)
    Et = (io_u == ti_col).astype(jnp.bfloat16)
    qr = jnp.dot(Et, qb, preferred_element_type=jnp.float32)
    sc = jax.lax.dot_general(
        qr.astype(jnp.bfloat16), kb, (((1,), (1,)), ((), ())),
        preferred_element_type=jnp.float32) * (1.0 / math.sqrt(_DH))
    mx = jnp.max(sc, axis=1, keepdims=True)
    ex = jnp.exp(sc - mx)
    attn = (ex / jnp.sum(ex, axis=1, keepdims=True)).astype(jnp.bfloat16)
    av = jnp.dot(attn, vb, preferred_element_type=jnp.float32)
    vmean = jnp.mean(v_ref[...].astype(jnp.float32), axis=0,
                     keepdims=True)
    delta = (av - vmean).astype(jnp.bfloat16)
    ctx = jnp.dot(E, delta, preferred_element_type=jnp.float32) + vmean
    o_ref[...] = ctx.astype(jnp.bfloat16)


def _wo_addnorm_kernel(c_ref, w_ref, b_ref, h_ref, o_ref):
    t = jnp.dot(c_ref[...], w_ref[...], preferred_element_type=jnp.float32)
    o_ref[...] = _ln(t + b_ref[...] + h_ref[...])


def _ffn1_kernel(y_ref, w_ref, b_ref, o_ref):
    t = jnp.dot(y_ref[...].astype(jnp.bfloat16), w_ref[...],
                preferred_element_type=jnp.float32)
    o_ref[...] = jnp.maximum(t + b_ref[...], 0.0).astype(jnp.bfloat16)


def _ffn2_kernel(r_ref, w_ref, b_ref, y_ref, o_ref):
    t = jnp.dot(r_ref[...], w_ref[...], preferred_element_type=jnp.float32)
    o_ref[...] = _ln(t + b_ref[...] + y_ref[...])


def _ffn2_final_kernel(r_ref, w_ref, b_ref, y_ref, g_ref, bb_ref, o_ref):
    t = jnp.dot(r_ref[...], w_ref[...], preferred_element_type=jnp.float32)
    hn = _ln(t + b_ref[...] + y_ref[...])
    o_ref[...] = _ln(hn) * g_ref[...] + bb_ref[...]


def _pad_cols(w):
    return jnp.pad(w.reshape(_DM, _H, _DH), ((0, 0), (0, 0), (0, _DP - _DH))
                   ).reshape(_DM, _HP)


def _pad_bias(b):
    return jnp.pad(b.reshape(_H, _DH), ((0, 0), (0, _DP - _DH))).reshape(_HP)


def kernel(x, Wq, bq, Wk, bk, Wv, bv, Wo, bo, W1, b1, W2, b2, gamma, beta):
    f32 = jnp.float32
    bf16 = jnp.bfloat16
    h = x.reshape(_BL, _DM)
    cnt_t = jnp.asarray(_CNT_T)
    neg_t = jnp.asarray(_NEG_T)

    out = None
    for i in range(2):
        wqkv = jnp.concatenate(
            [_pad_cols(Wq[i]), _pad_cols(Wk[i]), _pad_cols(Wv[i])],
            axis=1).astype(bf16)
        bqkv = jnp.concatenate(
            [_pad_bias(bq[i]), _pad_bias(bk[i]), _pad_bias(bv[i])])[None, :]
        wo_p = jnp.pad(Wo[i].reshape(_H, _DH, _DM),
                       ((0, 0), (0, _DP - _DH), (0, 0))
                       ).reshape(_HP, _DM).astype(bf16)

        qkv = pl.pallas_call(
            _mm_bias_kernel,
            grid=(16,),
            in_specs=[
                pl.BlockSpec((256, _DM), lambda m: (m, 0)),
                pl.BlockSpec((_DM, 3 * _HP), lambda m: (0, 0)),
                pl.BlockSpec((1, 3 * _HP), lambda m: (0, 0)),
            ],
            out_specs=pl.BlockSpec((256, 3 * _HP), lambda m: (m, 0)),
            out_shape=jax.ShapeDtypeStruct((_BL, 3 * _HP), bf16),
        )(h, wqkv, bqkv)

        M = pl.pallas_call(
            _mstats_kernel,
            grid=(_KT, _B, _H),
            in_specs=[
                pl.BlockSpec((_L, _DP), lambda kt, b, hh: (b, hh)),
                pl.BlockSpec((_KC, _DP),
                             lambda kt, b, hh: (b * _KT + kt, _H + hh)),
                pl.BlockSpec((_KC, _L), lambda kt, b, hh: (kt, 0)),
                pl.BlockSpec((_KC, _L), lambda kt, b, hh: (kt, 0)),
            ],
            out_specs=pl.BlockSpec((_BH, _L), lambda kt, b, hh: (0, 0)),
            out_shape=jax.ShapeDtypeStruct((_BH, _L), f32),
            scratch_shapes=[pltpu.VMEM((_BH, _L), f32),
                            pltpu.VMEM((_BH, _L), f32)],
        )(qkv, qkv, cnt_t, neg_t)

        TI = pl.pallas_call(
            _topk_kernel,
            in_specs=[pl.BlockSpec((_BH, _L), lambda: (0, 0))],
            out_specs=pl.BlockSpec((_BH, _U), lambda: (0, 0)),
            out_shape=jax.ShapeDtypeStruct((_BH, _U), jnp.int32),
        )(M)

        ti_row = TI.reshape(_BH, 1, _U)
        ti_col = TI.reshape(_BH, _U, 1)

        ctx = pl.pallas_call(
            _attn_kernel,
            grid=(_B, _H),
            in_specs=[
                pl.BlockSpec((_L, _DP), lambda b, hh: (b, hh)),
                pl.BlockSpec((_L, _DP), lambda b, hh: (b, _H + hh)),
                pl.BlockSpec((_L, _DP), lambda b, hh: (b, 2 * _H + hh)),
                pl.BlockSpec((1, 1, _U), lambda b, hh: (b * _H + hh, 0, 0)),
                pl.BlockSpec((1, _U, 1), lambda b, hh: (b * _H + hh, 0, 0)),
            ],
            out_specs=pl.BlockSpec((_L, _DP), lambda b, hh: (b, hh)),
            out_shape=jax.ShapeDtypeStruct((_BL, _HP), bf16),
        )(qkv, qkv, qkv, ti_row, ti_col)

        Y = pl.pallas_call(
            _wo_addnorm_kernel,
            grid=(16,),
            in_specs=[
                pl.BlockSpec((256, _HP), lambda m: (m, 0)),
                pl.BlockSpec((_HP, _DM), lambda m: (0, 0)),
                pl.BlockSpec((1, _DM), lambda m: (0, 0)),
                pl.BlockSpec((256, _DM), lambda m: (m, 0)),
            ],
            out_specs=pl.BlockSpec((256, _DM), lambda m: (m, 0)),
            out_shape=jax.ShapeDtypeStruct((_BL, _DM), f32),
        )(ctx, wo_p, bo[i][None, :], h)

        R = pl.pallas_call(
            _ffn1_kernel,
            grid=(8,),
            in_specs=[
                pl.BlockSpec((512, _DM), lambda m: (m, 0)),
                pl.BlockSpec((_DM, 4096), lambda m: (0, 0)),
                pl.BlockSpec((1, 4096), lambda m: (0, 0)),
            ],
            out_specs=pl.BlockSpec((512, 4096), lambda m: (m, 0)),
            out_shape=jax.ShapeDtypeStruct((_BL, 4096), bf16),
        )(Y, W1[i].astype(bf16), b1[i][None, :])

        if i == 0:
            h = pl.pallas_call(
                _ffn2_kernel,
                grid=(16,),
                in_specs=[
                    pl.BlockSpec((256, 4096), lambda m: (m, 0)),
                    pl.BlockSpec((4096, _DM), lambda m: (0, 0)),
                    pl.BlockSpec((1, _DM), lambda m: (0, 0)),
                    pl.BlockSpec((256, _DM), lambda m: (m, 0)),
                ],
                out_specs=pl.BlockSpec((256, _DM), lambda m: (m, 0)),
                out_shape=jax.ShapeDtypeStruct((_BL, _DM), f32),
            )(R, W2[i].astype(bf16), b2[i][None, :], Y)
        else:
            out = pl.pallas_call(
                _ffn2_final_kernel,
                grid=(16,),
                in_specs=[
                    pl.BlockSpec((256, 4096), lambda m: (m, 0)),
                    pl.BlockSpec((4096, _DM), lambda m: (0, 0)),
                    pl.BlockSpec((1, _DM), lambda m: (0, 0)),
                    pl.BlockSpec((256, _DM), lambda m: (m, 0)),
                    pl.BlockSpec((1, _DM), lambda m: (0, 0)),
                    pl.BlockSpec((1, _DM), lambda m: (0, 0)),
                ],
                out_specs=pl.BlockSpec((256, _DM), lambda m: (m, 0)),
                out_shape=jax.ShapeDtypeStruct((_BL, _DM), f32),
            )(R, W2[i].astype(bf16), b2[i][None, :], Y,
              gamma[None, :], beta[None, :])
    return out.reshape(_B, _L, _DM)
